# baseline (speedup 1.0000x reference)
import functools

import jax
import jax.numpy as jnp
from jax import lax
from jax.experimental import pallas as pl
from jax.experimental.pallas import tpu as pltpu
from jax.experimental.pallas import tpu_sc as plsc

N, E, D, H = 10000, 320000, 128, 8
DH = D // H
HD2 = D // 2

NTILES = 32
EPT_B = E // NTILES
EPT_D = E // 16
CHUNK_B = 400
CHUNK_D = 80
NCH_B = EPT_B // CHUNK_B
NCH_D = EPT_D // CHUNK_D
NROW = 624
NROW_LAST = N - 15 * NROW

_f32 = jnp.float32


def _ln(x, g, b):
    mu = jnp.mean(x, axis=-1, keepdims=True)
    var = jnp.mean((x - mu) ** 2, axis=-1, keepdims=True)
    return (x - mu) / jnp.sqrt(var + 1e-5) * g + b


def _node_pre_body(h_ref, qw_ref, qb_ref, kw_ref, kb_ref, vw_ref, vb_ref,
                   g1_ref, b1_ref, ga_ref, ba_ref, t_ref,
                   q_ref, k_ref, v_ref):
    h = h_ref[...]
    hn = _ln(h, g1_ref[...], b1_ref[...])
    ga, ba = ga_ref[...], ba_ref[...]
    scale = 1.0 / (4.0 * t_ref[0, 0])
    q_ref[...] = _ln(hn @ qw_ref[...] + qb_ref[...], ga, ba) * scale
    k_ref[...] = _ln(hn @ kw_ref[...] + kb_ref[...], ga, ba)
    v_ref[...] = _ln(hn @ vw_ref[...] + vb_ref[...], ga, ba)


def _node_pre(h, Qw, Qb, Kw, Kb, Vw, Vb, g1, b1, ga, ba, temp):
    TN = 1000
    grid = (N // TN,)
    row = pl.BlockSpec((1, D), lambda i: (0, 0))
    wspec = pl.BlockSpec((D, D), lambda i: (0, 0))
    nblk = pl.BlockSpec((TN, D), lambda i: (i, 0))
    return pl.pallas_call(
        _node_pre_body,
        grid=grid,
        in_specs=[nblk, wspec, row, wspec, row, wspec, row,
                  row, row, row, row,
                  pl.BlockSpec((1, 1), lambda i: (0, 0))],
        out_specs=[nblk, nblk, nblk],
        out_shape=[jax.ShapeDtypeStruct((N, D), _f32),
                   jax.ShapeDtypeStruct((N, D), _f32),
                   jax.ShapeDtypeStruct((N, D), _f32)],
    )(h, Qw, Qb.reshape(1, D), Kw, Kb.reshape(1, D), Vw, Vb.reshape(1, D),
      g1.reshape(1, D), b1.reshape(1, D), ga.reshape(1, D), ba.reshape(1, D),
      temp.reshape(1, 1))


def _gather_qk_body(k_hbm, q_hbm, ei_hbm, qk_hbm, idx, ks, qs, sem):
    wid = lax.axis_index("s") * 2 + lax.axis_index("c")
    tile_base = wid * EPT_B

    def chunk(c, carry):
        base = pl.multiple_of(tile_base + c * CHUNK_B, 8)
        pltpu.sync_copy(ei_hbm.at[:, pl.ds(base, CHUNK_B)], idx)
        cp1 = pltpu.async_copy(k_hbm.at[idx.at[0]], ks, sem)
        cp2 = pltpu.async_copy(q_hbm.at[idx.at[1]], qs, sem)
        cp1.wait()
        cp2.wait()

        @plsc.parallel_loop(0, CHUNK_B, 1, unroll=4)
        def _(i):
            for v in range(D // 16):
                sl = pl.ds(v * 16, 16)
                qs[i, sl] = ks[i, sl] * qs[i, sl]
        pltpu.sync_copy(qs, qk_hbm.at[pl.ds(base, CHUNK_B)])
        return carry

    lax.fori_loop(0, NCH_B, chunk, 0)


def _gather_qk(kh, qh, edge_index):
    mesh = plsc.VectorSubcoreMesh(core_axis_name="c", subcore_axis_name="s")
    fn = pl.kernel(
        _gather_qk_body,
        out_type=jax.ShapeDtypeStruct((E, D), _f32),
        mesh=mesh,
        compiler_params=pltpu.CompilerParams(use_tc_tiling_on_sc=False),
        scratch_types=[pltpu.VMEM((2, CHUNK_B), jnp.int32),
                       pltpu.VMEM((CHUNK_B, D), _f32),
                       pltpu.VMEM((CHUNK_B, D), _f32),
                       pltpu.SemaphoreType.DMA],
    )
    return fn(kh, qh, edge_index)


def _edge_plp_body(e_ref, sp_ref, pew_ref, peb_ref, posw_ref, posb_ref,
                   pemb_ref, g1e_ref, b1e_ref, ga_ref, ba_ref, plp_ref):
    ga, ba = ga_ref[...], ba_ref[...]
    en_ = _ln(e_ref[...], g1e_ref[...], b1e_ref[...])
    pe = _ln(en_ @ pew_ref[...] + peb_ref[...], ga, ba)
    lp = _ln(sp_ref[...] @ posw_ref[...] + posb_ref[...] + pemb_ref[...], ga, ba)
    plp_ref[...] = pe + lp


def _edge_plp(e, sp, Pew, Peb, Posw, Posb, pos_emb, g1e, b1e, ga, ba):
    TE = 2000
    grid = (E // TE,)
    row = pl.BlockSpec((1, D), lambda i: (0, 0))
    wspec = pl.BlockSpec((D, D), lambda i: (0, 0))
    blk = pl.BlockSpec((TE, D), lambda i: (i, 0))
    return pl.pallas_call(
        _edge_plp_body,
        grid=grid,
        in_specs=[blk, blk, wspec, row, wspec, row, row, row, row, row, row],
        out_specs=blk,
        out_shape=jax.ShapeDtypeStruct((E, D), _f32),
    )(e, sp, Pew, Peb.reshape(1, D), Posw, Posb.reshape(1, D), pos_emb,
      g1e.reshape(1, D), b1e.reshape(1, D), ga.reshape(1, D), ba.reshape(1, D))


def _edge_score_body(e_ref, qk_ref, plp_ref, oew_ref, oeb_ref, gmat_ref,
                     s_ref, e2_ref):
    p = jnp.exp(jnp.clip(qk_ref[...] + plp_ref[...], -5.0, 5.0))
    denom = p @ gmat_ref[...]
    score = p / denom
    s_ref[...] = score
    e2_ref[...] = score @ oew_ref[...] + oeb_ref[...] + e_ref[...]


def _edge_score(e, qk, plp, Oew, Oeb):
    TE = 2000
    grid = (E // TE,)
    gmat = jnp.kron(jnp.eye(H, dtype=_f32), jnp.ones((DH, DH), dtype=_f32))
    row = pl.BlockSpec((1, D), lambda i: (0, 0))
    wspec = pl.BlockSpec((D, D), lambda i: (0, 0))
    blk = pl.BlockSpec((TE, D), lambda i: (i, 0))
    return pl.pallas_call(
        _edge_score_body,
        grid=grid,
        in_specs=[blk, blk, blk, wspec, row, wspec],
        out_specs=[blk, blk],
        out_shape=[jax.ShapeDtypeStruct((E, D), _f32),
                   jax.ShapeDtypeStruct((E, D), _f32)],
    )(e, qk, plp, Oew, Oeb.reshape(1, D), gmat)


def _edge_ffn_body(e2_ref, w1_ref, b1_ref, w2_ref, b2_ref, g2e_ref, b2e_ref,
                   eo_ref):
    e2 = e2_ref[...]
    en2 = _ln(e2, g2e_ref[...], b2e_ref[...])
    t = jnp.maximum(en2 @ w1_ref[...] + b1_ref[...], 0.0)
    eo_ref[...] = e2 + (t @ w2_ref[...] + b2_ref[...])


def _edge_ffn(e2, w1, b1, w2, b2, g2e, b2e):
    TE = 2000
    grid = (E // TE,)
    row = pl.BlockSpec((1, D), lambda i: (0, 0))
    blk = pl.BlockSpec((TE, D), lambda i: (i, 0))
    return pl.pallas_call(
        _edge_ffn_body,
        grid=grid,
        in_specs=[blk,
                  pl.BlockSpec((D, 4 * D), lambda i: (0, 0)),
                  pl.BlockSpec((1, 4 * D), lambda i: (0, 0)),
                  pl.BlockSpec((4 * D, D), lambda i: (0, 0)),
                  row, row, row],
        out_specs=blk,
        out_shape=jax.ShapeDtypeStruct((E, D), _f32),
    )(e2, w1, b1.reshape(1, 4 * D), w2, b2.reshape(1, D),
      g2e.reshape(1, D), b2e.reshape(1, D))


def _segsum(v, score, edge_index):
    mesh = plsc.VectorSubcoreMesh(core_axis_name="c", subcore_axis_name="s")
    fn = pl.kernel(
        _segsum_body,
        out_type=jax.ShapeDtypeStruct((2, N, D), _f32),
        mesh=mesh,
        compiler_params=pltpu.CompilerParams(use_tc_tiling_on_sc=False),
        scratch_types=[pltpu.VMEM_SHARED((N, D), _f32),
                       [pltpu.VMEM((2, CHUNK_D), jnp.int32)] * 2,
                       [pltpu.VMEM((1, CHUNK_D), jnp.int32)] * 2,
                       [pltpu.VMEM((CHUNK_D, D), _f32)] * 2,
                       [pltpu.VMEM((CHUNK_D, D), _f32)] * 2,
                       [pltpu.SemaphoreType.DMA] * 2,
                       [pltpu.SemaphoreType.DMA] * 2],
    )
    return fn(v, score, edge_index)


def _segsum_body(v_hbm, s_hbm, ei_hbm, acc_hbm, acc,
                 idx, sidx, vs, scpay, gsem, ssem):
    cid = lax.axis_index("c")
    sid = lax.axis_index("s")

    def zinit(i, carry):
        for j in range(D // 16):
            scpay[0][i, pl.ds(j * 16, 16)] = jnp.zeros((16,), _f32)
        return carry

    lax.fori_loop(0, CHUNK_D, zinit, 0)
    row0 = sid * NROW
    nz = NROW // CHUNK_D
    for b in range(nz):
        pltpu.sync_copy(scpay[0], acc.at[pl.ds(row0 + b * CHUNK_D, CHUNK_D)])
    rem = NROW - nz * CHUNK_D
    pltpu.sync_copy(scpay[0].at[pl.ds(0, rem)],
                    acc.at[pl.ds(row0 + nz * CHUNK_D, rem)])

    @pl.when(sid == 15)
    def _():
        pltpu.sync_copy(scpay[0], acc.at[pl.ds(row0 + NROW_LAST - CHUNK_D,
                                               CHUNK_D)])

    plsc.subcore_barrier()

    tile_base = sid * EPT_D

    def prefetch(g, s):
        base = pl.multiple_of(tile_base + g * CHUNK_D, 8)
        pltpu.sync_copy(ei_hbm.at[:, pl.ds(base, CHUNK_D)], idx[s])
        pltpu.async_copy(v_hbm.at[idx[s].at[0]], vs[s], gsem[s])

    def run_mul(soff, s):
        @plsc.parallel_loop(0, CHUNK_D, 1, unroll=4)
        def _(i):
            for v in range(HD2 // 16):
                x = scpay[s][i, pl.ds(soff + v * 16, 16)]
                scpay[s][i, pl.ds(v * 16, 16)] = (
                    vs[s][i, pl.ds(soff + v * 16, 16)] * x)
                scpay[s][i, pl.ds(HD2 + v * 16, 16)] = x

    for s in range(2):
        prefetch(s, s)

    def pair(t, carry):
        for s in range(2):
            g = 2 * t + s

            @pl.when(t > 0)
            def _():
                pltpu.make_async_copy(s_hbm.at[pl.ds(0, CHUNK_D)],
                                      scpay[s], ssem[s]).wait()

            base = pl.multiple_of(tile_base + g * CHUNK_D, 8)
            pltpu.sync_copy(s_hbm.at[pl.ds(base, CHUNK_D)], scpay[s])
            pltpu.make_async_copy(s_hbm.at[pl.ds(0, CHUNK_D)],
                                  vs[s], gsem[s]).wait()

            @pl.when(cid == 0)
            def _():
                run_mul(0, s)

            @pl.when(cid == 1)
            def _():
                run_mul(HD2, s)

            for j in range(CHUNK_D // 16):
                sidx[s][0, pl.ds(j * 16, 16)] = idx[s][1, pl.ds(j * 16, 16)]
            pltpu.async_copy(scpay[s], acc.at[sidx[s].at[0]], ssem[s],
                             add=True)

            @pl.when(t < (NCH_D // 2) - 1)
            def _():
                prefetch(g + 2, s)

        return carry

    lax.fori_loop(0, NCH_D // 2, pair, 0)
    for s in range(2):
        pltpu.make_async_copy(s_hbm.at[pl.ds(0, CHUNK_D)],
                              scpay[s], ssem[s]).wait()
    plsc.subcore_barrier()

    @pl.when(sid < 15)
    def _():
        pltpu.sync_copy(acc.at[pl.ds(row0, NROW)],
                        acc_hbm.at[cid, pl.ds(row0, NROW)])

    @pl.when(sid == 15)
    def _():
        pltpu.sync_copy(acc.at[pl.ds(row0, NROW_LAST)],
                        acc_hbm.at[cid, pl.ds(row0, NROW_LAST)])


def _node_post_body(a0_ref, a1_ref, h_ref, ohw_ref, ohb_ref,
                    w1_ref, b1_ref, w2_ref, b2_ref, g2_ref, b2g_ref, ho_ref):
    a0 = a0_ref[0]
    a1 = a1_ref[0]
    wv = jnp.concatenate([a0[:, :HD2], a1[:, :HD2]], axis=1)
    z = jnp.concatenate([a0[:, HD2:], a1[:, HD2:]], axis=1)
    hout = wv / (z + 1e-6)
    h2 = hout @ ohw_ref[...] + ohb_ref[...] + h_ref[...]
    hn = _ln(h2, g2_ref[...], b2g_ref[...])
    t = jnp.maximum(hn @ w1_ref[...] + b1_ref[...], 0.0)
    ho_ref[...] = h2 + (t @ w2_ref[...] + b2_ref[...])


def _node_post(accs, h, Ohw, Ohb, w1, b1, w2, b2, g2, b2g):
    TN = 1000
    grid = (N // TN,)
    row = pl.BlockSpec((1, D), lambda i: (0, 0))
    return pl.pallas_call(
        _node_post_body,
        grid=grid,
        in_specs=[pl.BlockSpec((1, TN, D), lambda i: (0, i, 0)),
                  pl.BlockSpec((1, TN, D), lambda i: (1, i, 0)),
                  pl.BlockSpec((TN, D), lambda i: (i, 0)),
                  pl.BlockSpec((D, D), lambda i: (0, 0)), row,
                  pl.BlockSpec((D, 4 * D), lambda i: (0, 0)),
                  pl.BlockSpec((1, 4 * D), lambda i: (0, 0)),
                  pl.BlockSpec((4 * D, D), lambda i: (0, 0)), row,
                  row, row],
        out_specs=pl.BlockSpec((TN, D), lambda i: (i, 0)),
        out_shape=jax.ShapeDtypeStruct((N, D), _f32),
    )(accs, accs, h, Ohw, Ohb.reshape(1, D), w1, b1.reshape(1, 4 * D),
      w2, b2.reshape(1, D), g2.reshape(1, D), b2g.reshape(1, D))


def kernel(h, e, spatial_pos, edge_index, Qw, Qb, Kw, Kb, Vw, Vb, Pew, Peb,
           Posw, Posb, Ohw, Ohb, Oew, Oeb, ffnh_w1, ffnh_b1, ffnh_w2, ffnh_b2,
           ffne_w1, ffne_b1, ffne_w2, ffne_b2, pos_emb, temperature,
           ln_attn_g, ln_attn_b, ln1h_g, ln1h_b, ln1e_g, ln1e_b,
           ln2h_g, ln2h_b, ln2e_g, ln2e_b):
    qh, kh, vh = _node_pre(h, Qw, Qb, Kw, Kb, Vw, Vb,
                           ln1h_g, ln1h_b, ln_attn_g, ln_attn_b,
                           temperature)
    qk = _gather_qk(kh, qh, edge_index)
    plp = _edge_plp(e, spatial_pos, Pew, Peb, Posw, Posb, pos_emb,
                    ln1e_g, ln1e_b, ln_attn_g, ln_attn_b)
    score, e2 = _edge_score(e, qk, plp, Oew, Oeb)
    accs = _segsum(vh, score, edge_index)
    e_out = _edge_ffn(e2, ffne_w1, ffne_b1, ffne_w2, ffne_b2, ln2e_g, ln2e_b)
    h_out = _node_post(accs, h, Ohw, Ohb, ffnh_w1, ffnh_b1, ffnh_w2, ffnh_b2,
                       ln2h_g, ln2h_b)
    return (h_out, e_out)


# speedup vs baseline: 1.0182x; 1.0182x over previous
import functools

import jax
import jax.numpy as jnp
from jax import lax
from jax.experimental import pallas as pl
from jax.experimental.pallas import tpu as pltpu
from jax.experimental.pallas import tpu_sc as plsc

N, E, D, H = 10000, 320000, 128, 8
DH = D // H
HD2 = D // 2

NTILES = 32
EPT_B = E // NTILES
EPT_D = E // 16
CHUNK_B = 400
CHUNK_D = 160
NCH_B = EPT_B // CHUNK_B
NCH_D = EPT_D // CHUNK_D
NROW = 624
NROW_LAST = N - 15 * NROW

_f32 = jnp.float32


def _ln(x, g, b):
    mu = jnp.mean(x, axis=-1, keepdims=True)
    var = jnp.mean((x - mu) ** 2, axis=-1, keepdims=True)
    return (x - mu) / jnp.sqrt(var + 1e-5) * g + b


def _node_pre_body(h_ref, qw_ref, qb_ref, kw_ref, kb_ref, vw_ref, vb_ref,
                   g1_ref, b1_ref, ga_ref, ba_ref, t_ref,
                   q_ref, k_ref, v_ref):
    h = h_ref[...]
    hn = _ln(h, g1_ref[...], b1_ref[...])
    ga, ba = ga_ref[...], ba_ref[...]
    scale = 1.0 / (4.0 * t_ref[0, 0])
    q_ref[...] = _ln(hn @ qw_ref[...] + qb_ref[...], ga, ba) * scale
    k_ref[...] = _ln(hn @ kw_ref[...] + kb_ref[...], ga, ba)
    v_ref[...] = _ln(hn @ vw_ref[...] + vb_ref[...], ga, ba)


def _node_pre(h, Qw, Qb, Kw, Kb, Vw, Vb, g1, b1, ga, ba, temp):
    TN = 1000
    grid = (N // TN,)
    row = pl.BlockSpec((1, D), lambda i: (0, 0))
    wspec = pl.BlockSpec((D, D), lambda i: (0, 0))
    nblk = pl.BlockSpec((TN, D), lambda i: (i, 0))
    return pl.pallas_call(
        _node_pre_body,
        grid=grid,
        in_specs=[nblk, wspec, row, wspec, row, wspec, row,
                  row, row, row, row,
                  pl.BlockSpec((1, 1), lambda i: (0, 0))],
        out_specs=[nblk, nblk, nblk],
        out_shape=[jax.ShapeDtypeStruct((N, D), _f32),
                   jax.ShapeDtypeStruct((N, D), _f32),
                   jax.ShapeDtypeStruct((N, D), _f32)],
    )(h, Qw, Qb.reshape(1, D), Kw, Kb.reshape(1, D), Vw, Vb.reshape(1, D),
      g1.reshape(1, D), b1.reshape(1, D), ga.reshape(1, D), ba.reshape(1, D),
      temp.reshape(1, 1))


def _gather_qk_body(k_hbm, q_hbm, ei_hbm, qk_hbm, idx, ks, qs, sem):
    wid = lax.axis_index("s") * 2 + lax.axis_index("c")
    tile_base = wid * EPT_B

    def chunk(c, carry):
        base = pl.multiple_of(tile_base + c * CHUNK_B, 8)
        pltpu.sync_copy(ei_hbm.at[:, pl.ds(base, CHUNK_B)], idx)
        cp1 = pltpu.async_copy(k_hbm.at[idx.at[0]], ks, sem)
        cp2 = pltpu.async_copy(q_hbm.at[idx.at[1]], qs, sem)
        cp1.wait()
        cp2.wait()

        @plsc.parallel_loop(0, CHUNK_B, 1, unroll=4)
        def _(i):
            for v in range(D // 16):
                sl = pl.ds(v * 16, 16)
                qs[i, sl] = ks[i, sl] * qs[i, sl]
        pltpu.sync_copy(qs, qk_hbm.at[pl.ds(base, CHUNK_B)])
        return carry

    lax.fori_loop(0, NCH_B, chunk, 0)


def _gather_qk(kh, qh, edge_index):
    mesh = plsc.VectorSubcoreMesh(core_axis_name="c", subcore_axis_name="s")
    fn = pl.kernel(
        _gather_qk_body,
        out_type=jax.ShapeDtypeStruct((E, D), _f32),
        mesh=mesh,
        compiler_params=pltpu.CompilerParams(use_tc_tiling_on_sc=False),
        scratch_types=[pltpu.VMEM((2, CHUNK_B), jnp.int32),
                       pltpu.VMEM((CHUNK_B, D), _f32),
                       pltpu.VMEM((CHUNK_B, D), _f32),
                       pltpu.SemaphoreType.DMA],
    )
    return fn(kh, qh, edge_index)


def _edge_plp_body(e_ref, sp_ref, pew_ref, peb_ref, posw_ref, posb_ref,
                   pemb_ref, g1e_ref, b1e_ref, ga_ref, ba_ref, plp_ref):
    ga, ba = ga_ref[...], ba_ref[...]
    en_ = _ln(e_ref[...], g1e_ref[...], b1e_ref[...])
    pe = _ln(en_ @ pew_ref[...] + peb_ref[...], ga, ba)
    lp = _ln(sp_ref[...] @ posw_ref[...] + posb_ref[...] + pemb_ref[...], ga, ba)
    plp_ref[...] = pe + lp


def _edge_plp(e, sp, Pew, Peb, Posw, Posb, pos_emb, g1e, b1e, ga, ba):
    TE = 2000
    grid = (E // TE,)
    row = pl.BlockSpec((1, D), lambda i: (0, 0))
    wspec = pl.BlockSpec((D, D), lambda i: (0, 0))
    blk = pl.BlockSpec((TE, D), lambda i: (i, 0))
    return pl.pallas_call(
        _edge_plp_body,
        grid=grid,
        in_specs=[blk, blk, wspec, row, wspec, row, row, row, row, row, row],
        out_specs=blk,
        out_shape=jax.ShapeDtypeStruct((E, D), _f32),
    )(e, sp, Pew, Peb.reshape(1, D), Posw, Posb.reshape(1, D), pos_emb,
      g1e.reshape(1, D), b1e.reshape(1, D), ga.reshape(1, D), ba.reshape(1, D))


def _edge_score_body(e_ref, qk_ref, plp_ref, oew_ref, oeb_ref, gmat_ref,
                     s_ref, e2_ref):
    p = jnp.exp(jnp.clip(qk_ref[...] + plp_ref[...], -5.0, 5.0))
    denom = p @ gmat_ref[...]
    score = p / denom
    s_ref[...] = score
    e2_ref[...] = score @ oew_ref[...] + oeb_ref[...] + e_ref[...]


def _edge_score(e, qk, plp, Oew, Oeb):
    TE = 2000
    grid = (E // TE,)
    gmat = jnp.kron(jnp.eye(H, dtype=_f32), jnp.ones((DH, DH), dtype=_f32))
    row = pl.BlockSpec((1, D), lambda i: (0, 0))
    wspec = pl.BlockSpec((D, D), lambda i: (0, 0))
    blk = pl.BlockSpec((TE, D), lambda i: (i, 0))
    return pl.pallas_call(
        _edge_score_body,
        grid=grid,
        in_specs=[blk, blk, blk, wspec, row, wspec],
        out_specs=[blk, blk],
        out_shape=[jax.ShapeDtypeStruct((E, D), _f32),
                   jax.ShapeDtypeStruct((E, D), _f32)],
    )(e, qk, plp, Oew, Oeb.reshape(1, D), gmat)


def _edge_ffn_body(e2_ref, w1_ref, b1_ref, w2_ref, b2_ref, g2e_ref, b2e_ref,
                   eo_ref):
    e2 = e2_ref[...]
    en2 = _ln(e2, g2e_ref[...], b2e_ref[...])
    t = jnp.maximum(en2 @ w1_ref[...] + b1_ref[...], 0.0)
    eo_ref[...] = e2 + (t @ w2_ref[...] + b2_ref[...])


def _edge_ffn(e2, w1, b1, w2, b2, g2e, b2e):
    TE = 2000
    grid = (E // TE,)
    row = pl.BlockSpec((1, D), lambda i: (0, 0))
    blk = pl.BlockSpec((TE, D), lambda i: (i, 0))
    return pl.pallas_call(
        _edge_ffn_body,
        grid=grid,
        in_specs=[blk,
                  pl.BlockSpec((D, 4 * D), lambda i: (0, 0)),
                  pl.BlockSpec((1, 4 * D), lambda i: (0, 0)),
                  pl.BlockSpec((4 * D, D), lambda i: (0, 0)),
                  row, row, row],
        out_specs=blk,
        out_shape=jax.ShapeDtypeStruct((E, D), _f32),
    )(e2, w1, b1.reshape(1, 4 * D), w2, b2.reshape(1, D),
      g2e.reshape(1, D), b2e.reshape(1, D))


def _segsum(v, score, edge_index):
    mesh = plsc.VectorSubcoreMesh(core_axis_name="c", subcore_axis_name="s")
    fn = pl.kernel(
        _segsum_body,
        out_type=jax.ShapeDtypeStruct((2, N, D), _f32),
        mesh=mesh,
        compiler_params=pltpu.CompilerParams(use_tc_tiling_on_sc=False),
        scratch_types=[pltpu.VMEM_SHARED((N, D), _f32),
                       pltpu.VMEM((2, CHUNK_D), jnp.int32),
                       pltpu.VMEM((CHUNK_D, D), _f32),
                       pltpu.VMEM((CHUNK_D, D), _f32),
                       pltpu.SemaphoreType.DMA],
    )
    return fn(v, score, edge_index)


def _segsum_body(v_hbm, s_hbm, ei_hbm, acc_hbm, acc, idx, vs, scpay, sem):
    cid = lax.axis_index("c")
    sid = lax.axis_index("s")

    def zinit(i, carry):
        for j in range(D // 16):
            scpay[i, pl.ds(j * 16, 16)] = jnp.zeros((16,), _f32)
        return carry

    lax.fori_loop(0, CHUNK_D, zinit, 0)
    row0 = sid * NROW
    nz = NROW // CHUNK_D
    for b in range(nz):
        pltpu.sync_copy(scpay, acc.at[pl.ds(row0 + b * CHUNK_D, CHUNK_D)])
    rem = NROW - nz * CHUNK_D
    pltpu.sync_copy(scpay.at[pl.ds(0, rem)],
                    acc.at[pl.ds(row0 + nz * CHUNK_D, rem)])

    @pl.when(sid == 15)
    def _():
        pltpu.sync_copy(scpay, acc.at[pl.ds(row0 + NROW_LAST - CHUNK_D,
                                            CHUNK_D)])

    plsc.subcore_barrier()

    tile_base = sid * EPT_D

    def chunk(c, carry):
        base = pl.multiple_of(tile_base + c * CHUNK_D, 8)
        pltpu.sync_copy(ei_hbm.at[:, pl.ds(base, CHUNK_D)], idx)
        cp = pltpu.async_copy(v_hbm.at[idx.at[0]], vs, sem)
        pltpu.sync_copy(s_hbm.at[pl.ds(base, CHUNK_D)], scpay)
        cp.wait()

        def run_mul(soff):
            @plsc.parallel_loop(0, CHUNK_D, 1, unroll=4)
            def _(i):
                for v in range(HD2 // 16):
                    x = scpay[i, pl.ds(soff + v * 16, 16)]
                    scpay[i, pl.ds(v * 16, 16)] = (
                        vs[i, pl.ds(soff + v * 16, 16)] * x)
                    scpay[i, pl.ds(HD2 + v * 16, 16)] = x

        @pl.when(cid == 0)
        def _():
            run_mul(0)

        @pl.when(cid == 1)
        def _():
            run_mul(HD2)

        pltpu.sync_copy(scpay, acc.at[idx.at[1]], add=True)
        return carry

    lax.fori_loop(0, NCH_D, chunk, 0)
    plsc.subcore_barrier()

    @pl.when(sid < 15)
    def _():
        pltpu.sync_copy(acc.at[pl.ds(row0, NROW)],
                        acc_hbm.at[cid, pl.ds(row0, NROW)])

    @pl.when(sid == 15)
    def _():
        pltpu.sync_copy(acc.at[pl.ds(row0, NROW_LAST)],
                        acc_hbm.at[cid, pl.ds(row0, NROW_LAST)])


def _node_post_body(a0_ref, a1_ref, h_ref, ohw_ref, ohb_ref,
                    w1_ref, b1_ref, w2_ref, b2_ref, g2_ref, b2g_ref, ho_ref):
    a0 = a0_ref[0]
    a1 = a1_ref[0]
    wv = jnp.concatenate([a0[:, :HD2], a1[:, :HD2]], axis=1)
    z = jnp.concatenate([a0[:, HD2:], a1[:, HD2:]], axis=1)
    hout = wv / (z + 1e-6)
    h2 = hout @ ohw_ref[...] + ohb_ref[...] + h_ref[...]
    hn = _ln(h2, g2_ref[...], b2g_ref[...])
    t = jnp.maximum(hn @ w1_ref[...] + b1_ref[...], 0.0)
    ho_ref[...] = h2 + (t @ w2_ref[...] + b2_ref[...])


def _node_post(accs, h, Ohw, Ohb, w1, b1, w2, b2, g2, b2g):
    TN = 1000
    grid = (N // TN,)
    row = pl.BlockSpec((1, D), lambda i: (0, 0))
    return pl.pallas_call(
        _node_post_body,
        grid=grid,
        in_specs=[pl.BlockSpec((1, TN, D), lambda i: (0, i, 0)),
                  pl.BlockSpec((1, TN, D), lambda i: (1, i, 0)),
                  pl.BlockSpec((TN, D), lambda i: (i, 0)),
                  pl.BlockSpec((D, D), lambda i: (0, 0)), row,
                  pl.BlockSpec((D, 4 * D), lambda i: (0, 0)),
                  pl.BlockSpec((1, 4 * D), lambda i: (0, 0)),
                  pl.BlockSpec((4 * D, D), lambda i: (0, 0)), row,
                  row, row],
        out_specs=pl.BlockSpec((TN, D), lambda i: (i, 0)),
        out_shape=jax.ShapeDtypeStruct((N, D), _f32),
    )(accs, accs, h, Ohw, Ohb.reshape(1, D), w1, b1.reshape(1, 4 * D),
      w2, b2.reshape(1, D), g2.reshape(1, D), b2g.reshape(1, D))


def kernel(h, e, spatial_pos, edge_index, Qw, Qb, Kw, Kb, Vw, Vb, Pew, Peb,
           Posw, Posb, Ohw, Ohb, Oew, Oeb, ffnh_w1, ffnh_b1, ffnh_w2, ffnh_b2,
           ffne_w1, ffne_b1, ffne_w2, ffne_b2, pos_emb, temperature,
           ln_attn_g, ln_attn_b, ln1h_g, ln1h_b, ln1e_g, ln1e_b,
           ln2h_g, ln2h_b, ln2e_g, ln2e_b):
    qh, kh, vh = _node_pre(h, Qw, Qb, Kw, Kb, Vw, Vb,
                           ln1h_g, ln1h_b, ln_attn_g, ln_attn_b,
                           temperature)
    qk = _gather_qk(kh, qh, edge_index)
    plp = _edge_plp(e, spatial_pos, Pew, Peb, Posw, Posb, pos_emb,
                    ln1e_g, ln1e_b, ln_attn_g, ln_attn_b)
    score, e2 = _edge_score(e, qk, plp, Oew, Oeb)
    accs = _segsum(vh, score, edge_index)
    e_out = _edge_ffn(e2, ffne_w1, ffne_b1, ffne_w2, ffne_b2, ln2e_g, ln2e_b)
    h_out = _node_post(accs, h, Ohw, Ohb, ffnh_w1, ffnh_b1, ffnh_w2, ffnh_b2,
                       ln2h_g, ln2h_b)
    return (h_out, e_out)
